# baseline (speedup 1.0000x reference)
import functools

import jax
import jax.numpy as jnp
from jax import lax
from jax.experimental import pallas as pl
from jax.experimental.pallas import tpu as pltpu
from jax.experimental.pallas import tpu_sc as plsc

_NUM_USERS = 3000
_NUM_ITEMS = 7000
_N_NODES = _NUM_USERS + _NUM_ITEMS
_N_EDGES = 320000
_D = 128
_N_LAYERS = 3

_NC = 2
_NS = 16
_NW = _NC * _NS
_N_PAD = 10240
_E_TILE = _N_EDGES // _NW
_CHUNK = 80
_N_CHUNKS = _E_TILE // _CHUNK
_ROWS_TILE = _N_PAD // _NS
_ZROWS = 128


def _sc_layer_body(emb, src, dst, val, pout,
                   src_v0, src_v1, src_v2, dst_v0, dst_v1, dst_v2,
                   val_v0, val_v1, val_v2, dsc_v0, dsc_v1, dsc_v2,
                   rows_v0, rows_v1, rows_v2, zero_v, acc_sh,
                   sem_i0, sem_i1, sem_i2, sem_r0, sem_r1, sem_r2,
                   sem_s0, sem_s1, sem_s2):
  c = lax.axis_index("c")
  s = lax.axis_index("s")
  wid = s * _NC + c

  slot = [
      dict(sv=src_v0, dv=dst_v0, vv=val_v0, dvs=dsc_v0, rows=rows_v0,
           si=sem_i0, sr=sem_r0, ss=sem_s0),
      dict(sv=src_v1, dv=dst_v1, vv=val_v1, dvs=dsc_v1, rows=rows_v1,
           si=sem_i1, sr=sem_r1, ss=sem_s1),
      dict(sv=src_v2, dv=dst_v2, vv=val_v2, dvs=dsc_v2, rows=rows_v2,
           si=sem_i2, sr=sem_r2, ss=sem_s2),
  ]

  def issue_idx(g, b):
    base = wid * _E_TILE + g * _CHUNK
    pltpu.async_copy(src.at[pl.ds(base, _CHUNK)], b["sv"], b["si"])
    pltpu.async_copy(dst.at[pl.ds(base, _CHUNK)], b["dv"], b["si"])
    pltpu.async_copy(val.at[pl.ds(base, _CHUNK)], b["vv"], b["si"])

  def wait_idx(b):
    pltpu.make_async_copy(src.at[pl.ds(0, _CHUNK)], b["sv"], b["si"]).wait()
    pltpu.make_async_copy(dst.at[pl.ds(0, _CHUNK)], b["dv"], b["si"]).wait()
    pltpu.make_async_copy(val.at[pl.ds(0, _CHUNK)], b["vv"], b["si"]).wait()

  def issue_gather(b):
    pltpu.async_copy(emb.at[b["sv"]], b["rows"], b["sr"])

  def wait_gather(b):
    pltpu.make_async_copy(emb.at[b["sv"]], b["rows"], b["sr"]).wait()

  def issue_scat(b):
    pltpu.async_copy(b["rows"], acc_sh.at[b["dvs"]], b["ss"], add=True)

  def wait_scat(b):
    pltpu.make_async_copy(b["rows"], acc_sh.at[b["dvs"]], b["ss"]).wait()

  def scale(b):
    rows_v = b["rows"]
    val_v = b["vv"]

    def grp_body(g16, inner):
      v16 = val_v[pl.ds(g16 * 16, 16)]
      for jj in range(16):
        v = v16[jj]
        j = g16 * 16 + jj
        for k in range(_D // 16):
          sl = pl.ds(k * 16, 16)
          rows_v[j, sl] = rows_v[j, sl] * v
      return inner

    lax.fori_loop(0, _CHUNK // 16, grp_body, 0)

  issue_idx(0, slot[0])
  issue_idx(1, slot[1])
  issue_idx(2, slot[2])
  wait_idx(slot[0])
  issue_gather(slot[0])

  zeros16 = jnp.zeros((16,), jnp.float32)

  def zfill(i, carry):
    for k in range(_D // 16):
      zero_v[i, pl.ds(k * 16, 16)] = zeros16
    return carry

  lax.fori_loop(0, _ZROWS, zfill, 0)
  for z in range(_ROWS_TILE // _ZROWS):
    pltpu.sync_copy(zero_v, acc_sh.at[pl.ds(s * _ROWS_TILE + z * _ZROWS, _ZROWS)])
  plsc.subcore_barrier()

  def process(g, a, wait_prev, prefetch, idx_mode):
    cur, nxt = slot[a], slot[(a + 1) % 3]
    if wait_prev:
      wait_scat(nxt)
    if prefetch:
      wait_idx(nxt)
      issue_gather(nxt)
    wait_gather(cur)
    for q in range(_CHUNK // 16):
      sl = pl.ds(q * 16, 16)
      cur["dvs"][sl] = cur["dv"][sl]
    scale(cur)
    if idx_mode == 1:
      issue_idx(g + 3, cur)
    elif idx_mode == 2:
      @pl.when(g + 3 < _N_CHUNKS)
      def _():
        issue_idx(g + 3, cur)
    issue_scat(cur)

  process(0, 0, False, True, 1)
  process(1, 1, False, True, 1)
  process(2, 2, True, True, 1)

  def body3(t, carry):
    for i in range(3):
      process(3 * t + i, i, True, True, 2)
    return carry

  lax.fori_loop(1, (_N_CHUNKS - 2) // 3, body3, 0)
  process(_N_CHUNKS - 2, 0, True, True, 0)
  process(_N_CHUNKS - 1, 1, True, False, 0)
  wait_scat(slot[0])
  wait_scat(slot[1])
  plsc.subcore_barrier()

  for z in range(_ROWS_TILE // _ZROWS):
    r0 = s * _ROWS_TILE + z * _ZROWS
    pltpu.sync_copy(acc_sh.at[pl.ds(r0, _ZROWS)], pout.at[c, pl.ds(r0, _ZROWS)])


_sc_layer = pl.kernel(
    _sc_layer_body,
    out_type=jax.ShapeDtypeStruct((_NC, _N_PAD, _D), jnp.float32),
    mesh=plsc.VectorSubcoreMesh(
        core_axis_name="c", subcore_axis_name="s",
        num_cores=_NC, num_subcores=_NS),
    scratch_types=(
        [pltpu.VMEM((_CHUNK,), jnp.int32)] * 6
        + [pltpu.VMEM((_CHUNK,), jnp.float32)] * 3
        + [pltpu.VMEM((_CHUNK,), jnp.int32)] * 3
        + [pltpu.VMEM((_CHUNK, _D), jnp.float32)] * 3
        + [pltpu.VMEM((_ZROWS, _D), jnp.float32)]
        + [pltpu.VMEM_SHARED((_N_PAD, _D), jnp.float32)]
        + [pltpu.SemaphoreType.DMA] * 9
    ),
)


def _combine_body(p_ref, a_ref, e_ref, o_ref, *, scale):
  e = p_ref[0] + p_ref[1]
  e_ref[...] = e
  o_ref[...] = (a_ref[...] + e) * scale


_BR = 1024


def _combine(pout, acc, scale):
  grid = _N_PAD // _BR
  return pl.pallas_call(
      functools.partial(_combine_body, scale=scale),
      grid=(grid,),
      in_specs=[
          pl.BlockSpec((_NC, _BR, _D), lambda i: (0, i, 0)),
          pl.BlockSpec((_BR, _D), lambda i: (i, 0)),
      ],
      out_specs=[
          pl.BlockSpec((_BR, _D), lambda i: (i, 0)),
          pl.BlockSpec((_BR, _D), lambda i: (i, 0)),
      ],
      out_shape=[
          jax.ShapeDtypeStruct((_N_PAD, _D), jnp.float32),
          jax.ShapeDtypeStruct((_N_PAD, _D), jnp.float32),
      ],
  )(pout, acc)


def kernel(user_weight, item_weight, edge_index, edge_values):
  pad = jnp.zeros((_N_PAD - _N_NODES, _D), jnp.float32)
  emb = jnp.concatenate([user_weight, item_weight, pad], axis=0)
  src = edge_index[0]
  dst = edge_index[1]
  acc = emb
  for layer in range(_N_LAYERS):
    pout = _sc_layer(emb, src, dst, edge_values)
    scale = 0.25 if layer == _N_LAYERS - 1 else 1.0
    emb, acc = _combine(pout, acc, scale)
  return acc[:_NUM_USERS], acc[_NUM_USERS:_N_NODES]


# speedup vs baseline: 1.0051x; 1.0051x over previous
import functools

import jax
import jax.numpy as jnp
from jax import lax
from jax.experimental import pallas as pl
from jax.experimental.pallas import tpu as pltpu
from jax.experimental.pallas import tpu_sc as plsc

_NUM_USERS = 3000
_NUM_ITEMS = 7000
_N_NODES = _NUM_USERS + _NUM_ITEMS
_N_EDGES = 320000
_D = 128
_N_LAYERS = 3

_NC = 2
_NS = 16
_NW = _NC * _NS
_N_PAD = 10240
_E_TILE = _N_EDGES // _NW
_CHUNK = 80
_N_CHUNKS = _E_TILE // _CHUNK
_ROWS_TILE = _N_PAD // _NS
_ZROWS = 40


def _sc_layer_body(emb, src, dst, val, pout,
                   src_v0, src_v1, src_v2, src_v3,
                   dst_v0, dst_v1, dst_v2, dst_v3,
                   val_v0, val_v1, val_v2, val_v3,
                   dsc_v0, dsc_v1, dsc_v2, dsc_v3,
                   rows_v0, rows_v1, rows_v2, rows_v3, zero_v, acc_sh,
                   sem_i0, sem_i1, sem_i2, sem_i3,
                   sem_r0, sem_r1, sem_r2, sem_r3,
                   sem_s0, sem_s1, sem_s2, sem_s3):
  c = lax.axis_index("c")
  s = lax.axis_index("s")
  wid = s * _NC + c

  slot = [
      dict(sv=src_v0, dv=dst_v0, vv=val_v0, dvs=dsc_v0, rows=rows_v0,
           si=sem_i0, sr=sem_r0, ss=sem_s0),
      dict(sv=src_v1, dv=dst_v1, vv=val_v1, dvs=dsc_v1, rows=rows_v1,
           si=sem_i1, sr=sem_r1, ss=sem_s1),
      dict(sv=src_v2, dv=dst_v2, vv=val_v2, dvs=dsc_v2, rows=rows_v2,
           si=sem_i2, sr=sem_r2, ss=sem_s2),
      dict(sv=src_v3, dv=dst_v3, vv=val_v3, dvs=dsc_v3, rows=rows_v3,
           si=sem_i3, sr=sem_r3, ss=sem_s3),
  ]

  def issue_idx(g, b):
    base = wid * _E_TILE + g * _CHUNK
    pltpu.async_copy(src.at[pl.ds(base, _CHUNK)], b["sv"], b["si"])
    pltpu.async_copy(dst.at[pl.ds(base, _CHUNK)], b["dv"], b["si"])
    pltpu.async_copy(val.at[pl.ds(base, _CHUNK)], b["vv"], b["si"])

  def wait_idx(b):
    pltpu.make_async_copy(src.at[pl.ds(0, _CHUNK)], b["sv"], b["si"]).wait()
    pltpu.make_async_copy(dst.at[pl.ds(0, _CHUNK)], b["dv"], b["si"]).wait()
    pltpu.make_async_copy(val.at[pl.ds(0, _CHUNK)], b["vv"], b["si"]).wait()

  def issue_gather(b):
    pltpu.async_copy(emb.at[b["sv"]], b["rows"], b["sr"])

  def wait_gather(b):
    pltpu.make_async_copy(emb.at[b["sv"]], b["rows"], b["sr"]).wait()

  def issue_scat(b):
    pltpu.async_copy(b["rows"], acc_sh.at[b["dvs"]], b["ss"], add=True)

  def wait_scat(b):
    pltpu.make_async_copy(b["rows"], acc_sh.at[b["dvs"]], b["ss"]).wait()

  def scale(b):
    rows_v = b["rows"]
    val_v = b["vv"]

    def grp_body(g16, inner):
      v16 = val_v[pl.ds(g16 * 16, 16)]
      for jj in range(16):
        v = v16[jj]
        j = g16 * 16 + jj
        for k in range(_D // 16):
          sl = pl.ds(k * 16, 16)
          rows_v[j, sl] = rows_v[j, sl] * v
      return inner

    lax.fori_loop(0, _CHUNK // 16, grp_body, 0)

  issue_idx(0, slot[0])
  issue_idx(1, slot[1])
  issue_idx(2, slot[2])
  issue_idx(3, slot[3])
  wait_idx(slot[0])
  issue_gather(slot[0])
  wait_idx(slot[1])
  issue_gather(slot[1])

  zeros16 = jnp.zeros((16,), jnp.float32)

  def zfill(i, carry):
    for k in range(_D // 16):
      zero_v[i, pl.ds(k * 16, 16)] = zeros16
    return carry

  lax.fori_loop(0, _ZROWS, zfill, 0)
  for z in range(_ROWS_TILE // _ZROWS):
    pltpu.sync_copy(zero_v, acc_sh.at[pl.ds(s * _ROWS_TILE + z * _ZROWS, _ZROWS)])
  plsc.subcore_barrier()

  def process(g, a, wait_prev, prefetch, idx_mode):
    cur, pf = slot[a], slot[(a + 2) % 4]
    if wait_prev:
      wait_scat(pf)
    if prefetch:
      wait_idx(pf)
      issue_gather(pf)
    wait_gather(cur)
    for q in range(_CHUNK // 16):
      sl = pl.ds(q * 16, 16)
      cur["dvs"][sl] = cur["dv"][sl]
    scale(cur)
    if idx_mode == 1:
      issue_idx(g + 4, cur)
    elif idx_mode == 2:
      @pl.when(g + 4 < _N_CHUNKS)
      def _():
        issue_idx(g + 4, cur)
    issue_scat(cur)

  process(0, 0, False, True, 1)
  process(1, 1, False, True, 1)
  process(2, 2, True, True, 1)
  process(3, 3, True, True, 1)

  def body4(t, carry):
    for i in range(4):
      process(4 * t + i, i, True, True, 1)
    return carry

  lax.fori_loop(1, (_N_CHUNKS - 5) // 4, body4, 0)
  process(_N_CHUNKS - 5, 0, True, True, 1)
  process(_N_CHUNKS - 4, 1, True, True, 0)
  process(_N_CHUNKS - 3, 2, True, True, 0)
  process(_N_CHUNKS - 2, 3, True, False, 0)
  process(_N_CHUNKS - 1, 0, True, False, 0)
  wait_scat(slot[3])
  wait_scat(slot[0])
  plsc.subcore_barrier()

  for z in range(_ROWS_TILE // _ZROWS):
    r0 = s * _ROWS_TILE + z * _ZROWS
    pltpu.sync_copy(acc_sh.at[pl.ds(r0, _ZROWS)], pout.at[c, pl.ds(r0, _ZROWS)])


_sc_layer = pl.kernel(
    _sc_layer_body,
    out_type=jax.ShapeDtypeStruct((_NC, _N_PAD, _D), jnp.float32),
    mesh=plsc.VectorSubcoreMesh(
        core_axis_name="c", subcore_axis_name="s",
        num_cores=_NC, num_subcores=_NS),
    scratch_types=(
        [pltpu.VMEM((_CHUNK,), jnp.int32)] * 8
        + [pltpu.VMEM((_CHUNK,), jnp.float32)] * 4
        + [pltpu.VMEM((_CHUNK,), jnp.int32)] * 4
        + [pltpu.VMEM((_CHUNK, _D), jnp.float32)] * 4
        + [pltpu.VMEM((_ZROWS, _D), jnp.float32)]
        + [pltpu.VMEM_SHARED((_N_PAD, _D), jnp.float32)]
        + [pltpu.SemaphoreType.DMA] * 12
    ),
)


def _combine_body(p_ref, a_ref, e_ref, o_ref, *, scale):
  e = p_ref[0] + p_ref[1]
  e_ref[...] = e
  o_ref[...] = (a_ref[...] + e) * scale


_BR = 1024


def _combine(pout, acc, scale):
  grid = _N_PAD // _BR
  return pl.pallas_call(
      functools.partial(_combine_body, scale=scale),
      grid=(grid,),
      in_specs=[
          pl.BlockSpec((_NC, _BR, _D), lambda i: (0, i, 0)),
          pl.BlockSpec((_BR, _D), lambda i: (i, 0)),
      ],
      out_specs=[
          pl.BlockSpec((_BR, _D), lambda i: (i, 0)),
          pl.BlockSpec((_BR, _D), lambda i: (i, 0)),
      ],
      out_shape=[
          jax.ShapeDtypeStruct((_N_PAD, _D), jnp.float32),
          jax.ShapeDtypeStruct((_N_PAD, _D), jnp.float32),
      ],
  )(pout, acc)


def kernel(user_weight, item_weight, edge_index, edge_values):
  pad = jnp.zeros((_N_PAD - _N_NODES, _D), jnp.float32)
  emb = jnp.concatenate([user_weight, item_weight, pad], axis=0)
  src = edge_index[0]
  dst = edge_index[1]
  acc = emb
  for layer in range(_N_LAYERS):
    pout = _sc_layer(emb, src, dst, edge_values)
    scale = 0.25 if layer == _N_LAYERS - 1 else 1.0
    emb, acc = _combine(pout, acc, scale)
  return acc[:_NUM_USERS], acc[_NUM_USERS:_N_NODES]
